# Initial kernel scaffold; baseline (speedup 1.0000x reference)
import jax
import jax.numpy as jnp
from jax.experimental import pallas as pl


def kernel(batch_inputs, edge_list, edge_type, perm_idx, entity_embeddings, relation_embeddings, W_entities, a_h0, a2_h0, a_h1, a2_h1, W_rel, a_out, a2_out, Wg, bg, prelu_w, Wd, bd, rel_adj):
    raise NotImplementedError("write your pallas kernel here")


# speedup vs baseline: 3.7350x; 3.7350x over previous
import dataclasses
import functools
import jax
import jax.numpy as jnp
from jax import lax
from jax.experimental import pallas as pl
from jax.experimental.pallas import tpu as pltpu
from jax.experimental.pallas import tpu_sc as plsc

N = 10000
E = 320000
D = 128
NREL = 500
DCAT = 256
ALPHA = 0.2
BN = 2000
CH = 80
EPT = E // 16
NCHF = EPT // CH
NRELP = 512
RSR = 80


def _elu(v):
    return jnp.where(v > 0, v, jnp.exp(jnp.minimum(v, 0.0)) - 1.0)


def _tc_node_tables(X, A0h0T, A1h0T, A0h1T, A1h1T, v0, v1, Went):
    def body(x_ref, a00, a10, a01, a11, v0r, v1r, we,
             g1_ref, s0_ref, s1_ref, p0h_ref, upg_ref):
        x = x_ref[...]
        for h, (a0r, a1r, vr) in enumerate(((a00, a10, v0r), (a01, a11, v1r))):
            p0 = x @ a0r[...]
            p1 = x @ a1r[...]
            g1_ref[h] = p1
            s0_ref[h] = p0 @ vr[...]
            s1_ref[h] = p1 @ vr[...]
            p0h_ref[h] = p0
        upg_ref[...] = x @ we[...]

    wspec = pl.BlockSpec((D, D), lambda i: (0, 0))
    vspec = pl.BlockSpec((D, 1), lambda i: (0, 0))
    return pl.pallas_call(
        body,
        grid=(N // BN,),
        in_specs=[pl.BlockSpec((BN, D), lambda i: (i, 0)),
                  wspec, wspec, wspec, wspec, vspec, vspec,
                  pl.BlockSpec((D, DCAT), lambda i: (0, 0))],
        out_specs=[pl.BlockSpec((2, BN, D), lambda i: (0, i, 0)),
                   pl.BlockSpec((2, BN, 1), lambda i: (0, i, 0)),
                   pl.BlockSpec((2, BN, 1), lambda i: (0, i, 0)),
                   pl.BlockSpec((2, BN, D), lambda i: (0, i, 0)),
                   pl.BlockSpec((BN, DCAT), lambda i: (i, 0))],
        out_shape=[jax.ShapeDtypeStruct((2, N, D), jnp.float32),
                   jax.ShapeDtypeStruct((2, N, 1), jnp.float32),
                   jax.ShapeDtypeStruct((2, N, 1), jnp.float32),
                   jax.ShapeDtypeStruct((2, N, D), jnp.float32),
                   jax.ShapeDtypeStruct((N, DCAT), jnp.float32)],
    )(X, A0h0T, A1h0T, A0h1T, A1h1T, v0, v1, Went)


def _tc_rel_tables(Rtab, A2h0T, A2h1T, v0, v1, Wrel):
    def body(r_ref, a20, a21, v0r, v1r, wr, g1r_ref, sr_ref, orel_ref):
        r = r_ref[...]
        for h, (a2r, vr) in enumerate(((a20, v0r), (a21, v1r))):
            pr = r @ a2r[...]
            g1r_ref[h] = pr
            sr_ref[h] = (pr @ vr[...])[:, 0]
        orel_ref[...] = r @ wr[...]

    return pl.pallas_call(
        body,
        in_specs=[pl.BlockSpec((NREL, D), lambda: (0, 0)),
                  pl.BlockSpec((D, D), lambda: (0, 0)),
                  pl.BlockSpec((D, D), lambda: (0, 0)),
                  pl.BlockSpec((D, 1), lambda: (0, 0)),
                  pl.BlockSpec((D, 1), lambda: (0, 0)),
                  pl.BlockSpec((D, DCAT), lambda: (0, 0))],
        out_specs=[pl.BlockSpec((2, NREL, D), lambda: (0, 0, 0)),
                   pl.BlockSpec((2, NREL), lambda: (0, 0)),
                   pl.BlockSpec((NREL, DCAT), lambda: (0, 0))],
        out_shape=[jax.ShapeDtypeStruct((2, NREL, D), jnp.float32),
                   jax.ShapeDtypeStruct((2, NREL), jnp.float32),
                   jax.ShapeDtypeStruct((NREL, DCAT), jnp.float32)],
    )(Rtab, A2h0T, A2h1T, v0, v1, Wrel)


def _tc_combine(Hacc1, rs1, P0h, Ao0T, Ao1T, vo):
    def body(hacc_ref, rs_ref, p0h_ref, ao0, ao1, vr,
             g2_ref, s0o_ref, s1o_ref, p0o_ref):
        rs0 = rs_ref[0]
        h0 = (p0h_ref[0] * rs0 + hacc_ref[0]) / (rs0 + 1e-12)
        rs1v = rs_ref[1]
        h1 = (p0h_ref[1] * rs1v + hacc_ref[1]) / (rs1v + 1e-12)
        x = jnp.concatenate([_elu(h0), _elu(h1)], axis=1)
        p1 = x @ ao1[...]
        p0 = x @ ao0[...]
        g2_ref[0] = p1[:, 0:D]
        g2_ref[1] = p1[:, D:DCAT]
        s0 = p0 @ vr[...]
        s1 = p1 @ vr[...]
        s0o_ref[0] = s0
        s0o_ref[1] = s0
        s1o_ref[0] = s1
        s1o_ref[1] = s1
        p0o_ref[...] = p0

    return pl.pallas_call(
        body,
        grid=(N // BN,),
        in_specs=[pl.BlockSpec((2, BN, D), lambda i: (0, i, 0)),
                  pl.BlockSpec((2, BN, 1), lambda i: (0, i, 0)),
                  pl.BlockSpec((2, BN, D), lambda i: (0, i, 0)),
                  pl.BlockSpec((DCAT, DCAT), lambda i: (0, 0)),
                  pl.BlockSpec((DCAT, DCAT), lambda i: (0, 0)),
                  pl.BlockSpec((DCAT, 1), lambda i: (0, 0))],
        out_specs=[pl.BlockSpec((2, BN, D), lambda i: (0, i, 0)),
                   pl.BlockSpec((2, BN, 1), lambda i: (0, i, 0)),
                   pl.BlockSpec((2, BN, 1), lambda i: (0, i, 0)),
                   pl.BlockSpec((BN, DCAT), lambda i: (i, 0))],
        out_shape=[jax.ShapeDtypeStruct((2, N, D), jnp.float32),
                   jax.ShapeDtypeStruct((2, N, 1), jnp.float32),
                   jax.ShapeDtypeStruct((2, N, 1), jnp.float32),
                   jax.ShapeDtypeStruct((N, DCAT), jnp.float32)],
    )(Hacc1, rs1, P0h, Ao0T, Ao1T, vo)


def _tc_rel2(orel, Ao2T, vo):
    def body(orel_ref, ao2, vr, g2r_ref, sro_ref):
        pr = orel_ref[...] @ ao2[...]
        g2r_ref[0] = pr[:, 0:D]
        g2r_ref[1] = pr[:, D:DCAT]
        sr = (pr @ vr[...])[:, 0]
        sro_ref[0] = sr
        sro_ref[1] = sr

    return pl.pallas_call(
        body,
        in_specs=[pl.BlockSpec((NREL, DCAT), lambda: (0, 0)),
                  pl.BlockSpec((DCAT, DCAT), lambda: (0, 0)),
                  pl.BlockSpec((DCAT, 1), lambda: (0, 0))],
        out_specs=[pl.BlockSpec((2, NREL, D), lambda: (0, 0, 0)),
                   pl.BlockSpec((2, NREL), lambda: (0, 0))],
        out_shape=[jax.ShapeDtypeStruct((2, NREL, D), jnp.float32),
                   jax.ShapeDtypeStruct((2, NREL), jnp.float32)],
    )(orel, Ao2T, vo)


def _tc_final(Hacc2, rs2, P0o, upg, maskc):
    def body(hacc_ref, rs_ref, p0o_ref, upg_ref, m_ref, oe_ref):
        rs = rs_ref[...]
        lo = p0o_ref[:, 0:D] * rs + hacc_ref[0]
        hi = p0o_ref[:, D:DCAT] * rs + hacc_ref[1]
        h2 = jnp.concatenate([lo, hi], axis=1) / (rs + 1e-12)
        oe = _elu(h2)
        m = jnp.minimum(m_ref[...], 1.0)
        o = upg_ref[...] + m * oe
        nrm = jnp.sqrt(jnp.sum(o * o, axis=1, keepdims=True))
        oe_ref[...] = o / jnp.maximum(nrm, 1e-12)

    return pl.pallas_call(
        body,
        grid=(N // BN,),
        in_specs=[pl.BlockSpec((2, BN, D), lambda i: (0, i, 0)),
                  pl.BlockSpec((BN, 1), lambda i: (i, 0)),
                  pl.BlockSpec((BN, DCAT), lambda i: (i, 0)),
                  pl.BlockSpec((BN, DCAT), lambda i: (i, 0)),
                  pl.BlockSpec((BN, 1), lambda i: (i, 0))],
        out_specs=pl.BlockSpec((BN, DCAT), lambda i: (i, 0)),
        out_shape=jax.ShapeDtypeStruct((N, DCAT), jnp.float32),
    )(Hacc2, rs2, P0o, upg, maskc)


def _tc_dgi(orel, perm2d, rel_adj, Wg, bg2d, pw2d, Wd, bd2d):
    def body(orel_ref, perm_ref, adj_ref, wg_ref, bg_ref, pw_ref, wd_ref,
             bd_ref, h1_ref, scs_ref):
        orel_v = orel_ref[...]
        adj = adj_ref[...]
        wg = wg_ref[...]
        wd = wd_ref[...]
        bg = bg_ref[...]
        pw = pw_ref[...]
        bd = bd_ref[...][:, 0]

        def gcn(seq):
            o = adj @ (seq @ wg) + bg
            return jnp.where(o >= 0, o, pw * o)

        h1 = gcn(orel_v)
        iot = lax.broadcasted_iota(jnp.int32, (NREL, NREL), 1)
        oh = (perm_ref[...] == iot).astype(jnp.float32)
        h2 = gcn(oh @ orel_v)
        c = 1.0 / (1.0 + jnp.exp(-jnp.mean(orel_v, axis=0, keepdims=True)))
        scs_ref[0] = jnp.sum((h1 @ wd) * c, axis=1) + bd
        scs_ref[1] = jnp.sum((h2 @ wd) * c, axis=1) + bd
        h1_ref[...] = h1

    return pl.pallas_call(
        body,
        in_specs=[pl.BlockSpec((NREL, DCAT), lambda: (0, 0)),
                  pl.BlockSpec((NREL, 1), lambda: (0, 0)),
                  pl.BlockSpec((NREL, NREL), lambda: (0, 0)),
                  pl.BlockSpec((DCAT, DCAT), lambda: (0, 0)),
                  pl.BlockSpec((1, DCAT), lambda: (0, 0)),
                  pl.BlockSpec((1, 1), lambda: (0, 0)),
                  pl.BlockSpec((DCAT, DCAT), lambda: (0, 0)),
                  pl.BlockSpec((1, 1), lambda: (0, 0))],
        out_specs=[pl.BlockSpec((NREL, DCAT), lambda: (0, 0)),
                   pl.BlockSpec((2, NREL), lambda: (0, 0))],
        out_shape=[jax.ShapeDtypeStruct((NREL, DCAT), jnp.float32),
                   jax.ShapeDtypeStruct((2, NREL), jnp.float32)],
    )(orel, perm2d, rel_adj, Wg, bg2d, pw2d, Wd, bd2d)


@functools.cache
def _make_sc_phase(with_mask):
    mesh = plsc.VectorSubcoreMesh(core_axis_name="c", subcore_axis_name="s",
                                  num_cores=2, num_subcores=16)
    out_type = [jax.ShapeDtypeStruct((2 * N, D), jnp.float32),
                jax.ShapeDtypeStruct((2 * RSR, 128), jnp.float32)]
    if with_mask:
        out_type.append(jax.ShapeDtypeStruct((RSR, 128), jnp.float32))
    scratch = [
        pltpu.VMEM_SHARED((N, D), jnp.float32),
        pltpu.VMEM_SHARED((RSR, 128), jnp.float32),
        pltpu.VMEM((N,), jnp.float32),
        pltpu.VMEM((N,), jnp.float32),
        pltpu.VMEM((NRELP,), jnp.float32),
        pltpu.VMEM((CH,), jnp.int32),
        pltpu.VMEM((CH,), jnp.int32),
        pltpu.VMEM((CH,), jnp.int32),
        pltpu.VMEM((CH,), jnp.float32),
        pltpu.VMEM((CH, D), jnp.float32),
        pltpu.VMEM((RSR, 128), jnp.float32),
    ]
    if with_mask:
        scratch.append(pltpu.VMEM_SHARED((RSR, 128), jnp.float32))

    cp = pltpu.CompilerParams()
    if "needs_layout_passes" in pltpu.CompilerParams.__dataclass_fields__:
        cp = dataclasses.replace(cp, needs_layout_passes=False)

    @functools.partial(pl.kernel, mesh=mesh, out_type=tuple(out_type),
                       scratch_types=scratch, compiler_params=cp)
    def sc_phase(*refs):
        (e0_h, e1_h, et_h, g_h, gr_h, s0_h, s1_h, sr_h) = refs[:8]
        refs = refs[8:]
        if with_mask:
            tails_h = refs[0]
            hacc_o, rs_o, mask_o = refs[1], refs[2], refs[3]
            refs = refs[4:]
        else:
            hacc_o, rs_o = refs[0], refs[1]
            refs = refs[2:]
        (acc_sh, rs_sh, s0_v, s1_v, sr_v, e0b, e1b, etb,
         ee_v, rows_v, rsum_v) = refs[:11]
        if with_mask:
            mask_sh = refs[11]

        c = lax.axis_index("c")
        s = lax.axis_index("s")
        coff_n = c * N
        coff_r = c * NREL
        tbase = s * EPT
        sbase = s * 624
        stripes = [(k * 80, 80) for k in range(7)] + [(560, 64)]
        zeros16 = jnp.zeros((16,), jnp.float32)

        pltpu.sync_copy(s0_h.at[pl.ds(coff_n, N)], s0_v)
        pltpu.sync_copy(s1_h.at[pl.ds(coff_n, N)], s1_v)
        pltpu.sync_copy(sr_h.at[pl.ds(c * NRELP, NRELP)], sr_v)

        @pl.loop(0, CH)
        def _zero_rows(r):
            for j in range(D // 16):
                rows_v[r, pl.ds(j * 16, 16)] = zeros16

        @pl.loop(0, RSR)
        def _zero_rsum(r):
            for j in range(128 // 16):
                rsum_v[r, pl.ds(j * 16, 16)] = zeros16

        for off, sz in stripes:
            pltpu.sync_copy(rows_v.at[pl.ds(0, sz)],
                            acc_sh.at[pl.ds(sbase + off, sz)])

        @pl.when(s == 0)
        def _zero_tail():
            pltpu.sync_copy(rows_v.at[pl.ds(0, 16)],
                            acc_sh.at[pl.ds(9984, 16)])
            pltpu.sync_copy(rsum_v, rs_sh)

        if with_mask:
            @pl.when((c == 0) & (s == 0))
            def _zero_mask_sh():
                pltpu.sync_copy(rsum_v, mask_sh)

        plsc.subcore_barrier()

        @pl.loop(0, NCHF)
        def _main(k):
            base = tbase + k * CH
            pltpu.sync_copy(e0_h.at[pl.ds(base, CH)], e0b)
            pltpu.sync_copy(e1_h.at[pl.ds(base, CH)], e1b)
            pltpu.sync_copy(et_h.at[pl.ds(base, CH)], etb)

            @pl.loop(0, CH // 16)
            def _ee(i):
                sl = pl.ds(i * 16, 16)
                i0 = e0b[sl]
                i1 = e1b[sl]
                ir = etb[sl]
                v0 = plsc.load_gather(s0_v, [i0])
                v1 = plsc.load_gather(s1_v, [i1])
                vr = plsc.load_gather(sr_v, [ir])
                z = v0 + v1 + vr
                p = jnp.where(z >= 0, z, ALPHA * z)
                ee = jnp.exp(-p)
                ee_v[sl] = ee
                plsc.addupdate_scatter(
                    rsum_v, [lax.shift_right_logical(i0, 7), i0 & 127], ee)
                e1b[sl] = i1 + coff_n
                etb[sl] = ir + coff_r

            pltpu.sync_copy(g_h.at[e1b], rows_v)
            pltpu.sync_copy(gr_h.at[etb], rows_v, add=True)

            @pl.loop(0, CH // 16)
            def _scale(i):
                eev = ee_v[pl.ds(i * 16, 16)]
                for t in range(16):
                    sc = eev[t]
                    r = i * 16 + t
                    for j in range(D // 16):
                        sl = pl.ds(j * 16, 16)
                        rows_v[r, sl] = rows_v[r, sl] * sc

            pltpu.sync_copy(rows_v, acc_sh.at[e0b], add=True)

        @pl.loop(0, RSR // 16)
        def _fill_idn(k):
            e0b[pl.ds(k * 16, 16)] = lax.iota(jnp.int32, 16) + k * 16

        pltpu.sync_copy(rsum_v, rs_sh.at[e0b], add=True)

        if with_mask:
            @pl.when(c == 0)
            def _mask_scatter():
                @pl.loop(0, RSR)
                def _zm(r):
                    for j in range(128 // 16):
                        rsum_v[r, pl.ds(j * 16, 16)] = zeros16

                mbase = s * (8192 // 16)
                for k in range(8):
                    pltpu.sync_copy(tails_h.at[pl.ds(mbase + k * 64, 64)],
                                    e1b.at[pl.ds(0, 64)])

                    @pl.loop(0, 4)
                    def _mk(i):
                        ti = e1b[pl.ds(i * 16, 16)]
                        plsc.store_scatter(
                            rsum_v,
                            [lax.shift_right_logical(ti, 7), ti & 127],
                            jnp.ones((16,), jnp.float32))

                pltpu.sync_copy(rsum_v, mask_sh.at[e0b], add=True)

        plsc.subcore_barrier()

        for off, sz in stripes:
            pltpu.sync_copy(acc_sh.at[pl.ds(sbase + off, sz)],
                            hacc_o.at[pl.ds(coff_n + sbase + off, sz)])

        @pl.when(s == 0)
        def _out_tail():
            pltpu.sync_copy(acc_sh.at[pl.ds(9984, 16)],
                            hacc_o.at[pl.ds(coff_n + 9984, 16)])
            pltpu.sync_copy(rs_sh, rs_o.at[pl.ds(c * RSR, RSR)])

        if with_mask:
            @pl.when((c == 0) & (s == 0))
            def _mask_out():
                pltpu.sync_copy(mask_sh, mask_o)

    return sc_phase


def kernel(batch_inputs, edge_list, edge_type, perm_idx, entity_embeddings,
           relation_embeddings, W_entities, a_h0, a2_h0, a_h1, a2_h1, W_rel,
           a_out, a2_out, Wg, bg, prelu_w, Wd, bd, rel_adj):
    e0 = edge_list[0]
    e1 = edge_list[1]
    et = edge_type
    tails = batch_inputs[:, 2]

    A0h0T = a_h0[:, 0:D].T
    A1h0T = a_h0[:, D:2 * D].T
    A2h0T = a_h0[:, 2 * D:].T
    A0h1T = a_h1[:, 0:D].T
    A1h1T = a_h1[:, D:2 * D].T
    A2h1T = a_h1[:, 2 * D:].T
    Ao0T = a_out[:, 0:DCAT].T
    Ao1T = a_out[:, DCAT:2 * DCAT].T
    Ao2T = a_out[:, 2 * DCAT:].T
    v0 = a2_h0.T
    v1 = a2_h1.T
    vo = a2_out.T

    G1, S0, S1, P0h, upg = _tc_node_tables(
        entity_embeddings, A0h0T, A1h0T, A0h1T, A1h1T, v0, v1, W_entities)
    G1R, SR, orel = _tc_rel_tables(
        relation_embeddings, A2h0T, A2h1T, v0, v1, W_rel)

    SRp = jnp.pad(SR, ((0, 0), (0, NRELP - NREL))).reshape(-1)
    hacc1, rs1 = _make_sc_phase(False)(
        e0, e1, et, G1.reshape(2 * N, D), G1R.reshape(2 * NREL, D),
        S0.reshape(-1), S1.reshape(-1), SRp)
    rs1n = rs1.reshape(2, RSR * 128)[:, :N, None]

    G2, S0o, S1o, P0o = _tc_combine(hacc1.reshape(2, N, D), rs1n, P0h,
                                    Ao0T, Ao1T, vo)
    G2R, SRo = _tc_rel2(orel, Ao2T, vo)

    SRop = jnp.pad(SRo, ((0, 0), (0, NRELP - NREL))).reshape(-1)
    hacc2, rs2, mask = _make_sc_phase(True)(
        e0, e1, et, G2.reshape(2 * N, D), G2R.reshape(2 * NREL, D),
        S0o.reshape(-1), S1o.reshape(-1), SRop, tails)
    rs2n = rs2.reshape(2, RSR * 128)[0, :N, None]
    maskc = mask.reshape(-1)[:N, None]

    out_entity_1 = _tc_final(hacc2.reshape(2, N, D), rs2n, P0o, upg, maskc)
    h_1, scs = _tc_dgi(orel, perm_idx[:, None], rel_adj, Wg,
                       bg[None, :], prelu_w[None, None], Wd, bd[None, None])
    return (out_entity_1, h_1, scs.reshape(2 * NREL))
